# baseline (device time: 210641 ns/iter reference)
import jax
import jax.numpy as jnp
from jax import lax
from jax.experimental import pallas as pl
from jax.experimental.pallas import tpu as pltpu

T = 2048
D = 4096
V_SHARD = 8192
TILE_V = 512
N_TILES = V_SHARD // TILE_V


def kernel(x, W, labels):
    labels2d = labels.reshape(T, 1)

    def body(
        x_ref,
        w_ref,
        lab_ref,
        out_ref,
        s_acc,
        g_acc,
        s_rem,
        g_rem,
        send_sems,
        recv_sems,
    ):
        j = pl.program_id(0)
        my_x = lax.axis_index("x")
        my_y = lax.axis_index("y")
        nbr = (1 - my_x, my_y)

        @pl.when(j == 0)
        def _():
            barrier_sem = pltpu.get_barrier_semaphore()
            pl.semaphore_signal(
                barrier_sem,
                inc=1,
                device_id=nbr,
                device_id_type=pl.DeviceIdType.MESH,
            )
            pl.semaphore_wait(barrier_sem, 1)
            s_acc[:, :] = jnp.zeros((T, 1), dtype=jnp.float32)
            g_acc[:, :] = jnp.zeros((T, 1), dtype=jnp.float32)

        logits = jnp.dot(
            x_ref[:, :], w_ref[:, :], preferred_element_type=jnp.float32
        )

        e = jnp.exp(logits)

        col0 = my_x * V_SHARD + j * TILE_V
        lab_shift = lab_ref[:, :] - col0
        cols = lax.broadcasted_iota(jnp.int32, (T, TILE_V), 1)
        masked = jnp.where(cols == lab_shift, logits, 0.0)

        ones = jnp.ones((TILE_V, 128), dtype=jnp.float32)
        s_acc[:, :] = s_acc[:, :] + jnp.dot(
            e, ones, preferred_element_type=jnp.float32
        )[:, 0:1]
        g_acc[:, :] = g_acc[:, :] + jnp.dot(
            masked, ones, preferred_element_type=jnp.float32
        )[:, 0:1]

        @pl.when(j == N_TILES - 1)
        def _():
            copies = []
            for k, (src, dst) in enumerate(
                ((s_acc, s_rem), (g_acc, g_rem))
            ):
                c = pltpu.make_async_remote_copy(
                    src_ref=src,
                    dst_ref=dst,
                    send_sem=send_sems.at[k],
                    recv_sem=recv_sems.at[k],
                    device_id=nbr,
                    device_id_type=pl.DeviceIdType.MESH,
                )
                c.start()
                copies.append(c)
            for c in copies:
                c.wait()

            s_g = s_acc[:, :] + s_rem[:, :]
            g_g = g_acc[:, :] + g_rem[:, :]
            out_ref[:, :] = jnp.log(s_g) - g_g

    out = pl.pallas_call(
        body,
        grid=(N_TILES,),
        out_shape=jax.ShapeDtypeStruct((T, 1), jnp.float32),
        in_specs=[
            pl.BlockSpec((T, D), lambda j: (0, 0)),
            pl.BlockSpec((D, TILE_V), lambda j: (0, j)),
            pl.BlockSpec((T, 1), lambda j: (0, 0)),
        ],
        out_specs=pl.BlockSpec((T, 1), lambda j: (0, 0)),
        scratch_shapes=[
            pltpu.VMEM((T, 1), jnp.float32),
            pltpu.VMEM((T, 1), jnp.float32),
            pltpu.VMEM((T, 1), jnp.float32),
            pltpu.VMEM((T, 1), jnp.float32),
            pltpu.SemaphoreType.DMA((2,)),
            pltpu.SemaphoreType.DMA((2,)),
        ],
        compiler_params=pltpu.CompilerParams(
            collective_id=0,
            dimension_semantics=("arbitrary",),
            vmem_limit_bytes=100 * 1024 * 1024,
        ),
    )(x, W, labels2d)
    return out.reshape(T)


# device time: 209411 ns/iter; 1.0059x vs baseline; 1.0059x over previous
import jax
import jax.numpy as jnp
from jax import lax
from jax.experimental import pallas as pl
from jax.experimental.pallas import tpu as pltpu

T = 2048
D = 4096
V_SHARD = 8192
TILE_V = 512
N_TILES = V_SHARD // TILE_V


def kernel(x, W, labels):
    labels2d = labels.reshape(T, 1)

    def body(
        x_ref,
        w_ref,
        lab_ref,
        out_ref,
        s_acc,
        g_acc,
        s_send,
        g_send,
        s_rem,
        g_rem,
        send_sems,
        recv_sems,
    ):
        j = pl.program_id(0)
        my_x = lax.axis_index("x")
        my_y = lax.axis_index("y")
        nbr = (1 - my_x, my_y)

        @pl.when(j == 0)
        def _():
            barrier_sem = pltpu.get_barrier_semaphore()
            pl.semaphore_signal(
                barrier_sem,
                inc=1,
                device_id=nbr,
                device_id_type=pl.DeviceIdType.MESH,
            )
            pl.semaphore_wait(barrier_sem, 1)
            s_acc[:, :] = jnp.zeros((T, 128), dtype=jnp.float32)
            g_acc[:, :] = jnp.zeros((T, 128), dtype=jnp.float32)

        logits = jnp.dot(
            x_ref[:, :], w_ref[:, :], preferred_element_type=jnp.float32
        )

        e = jnp.exp(logits)

        col0 = my_x * V_SHARD + j * TILE_V
        lab_shift = lab_ref[:, :] - col0
        cols = lax.broadcasted_iota(jnp.int32, (T, TILE_V), 1)
        masked = jnp.where(cols == lab_shift, logits, 0.0)

        ones = jnp.ones((TILE_V, 128), dtype=jnp.float32)
        s_acc[:, :] = s_acc[:, :] + jnp.dot(
            e, ones, preferred_element_type=jnp.float32
        )
        g_acc[:, :] = g_acc[:, :] + jnp.dot(
            masked, ones, preferred_element_type=jnp.float32
        )

        @pl.when(j == N_TILES - 1)
        def _():
            s_send[:, :] = s_acc[:, 0:1]
            g_send[:, :] = g_acc[:, 0:1]
            copies = []
            for k, (src, dst) in enumerate(
                ((s_send, s_rem), (g_send, g_rem))
            ):
                c = pltpu.make_async_remote_copy(
                    src_ref=src,
                    dst_ref=dst,
                    send_sem=send_sems.at[k],
                    recv_sem=recv_sems.at[k],
                    device_id=nbr,
                    device_id_type=pl.DeviceIdType.MESH,
                )
                c.start()
                copies.append(c)
            for c in copies:
                c.wait()

            s_g = s_send[:, :] + s_rem[:, :]
            g_g = g_send[:, :] + g_rem[:, :]
            out_ref[:, :] = jnp.log(s_g) - g_g

    out = pl.pallas_call(
        body,
        grid=(N_TILES,),
        out_shape=jax.ShapeDtypeStruct((T, 1), jnp.float32),
        in_specs=[
            pl.BlockSpec((T, D), lambda j: (0, 0)),
            pl.BlockSpec((D, TILE_V), lambda j: (0, j)),
            pl.BlockSpec((T, 1), lambda j: (0, 0)),
        ],
        out_specs=pl.BlockSpec((T, 1), lambda j: (0, 0)),
        scratch_shapes=[
            pltpu.VMEM((T, 128), jnp.float32),
            pltpu.VMEM((T, 128), jnp.float32),
            pltpu.VMEM((T, 1), jnp.float32),
            pltpu.VMEM((T, 1), jnp.float32),
            pltpu.VMEM((T, 1), jnp.float32),
            pltpu.VMEM((T, 1), jnp.float32),
            pltpu.SemaphoreType.DMA((2,)),
            pltpu.SemaphoreType.DMA((2,)),
        ],
        compiler_params=pltpu.CompilerParams(
            collective_id=0,
            dimension_semantics=("arbitrary",),
            vmem_limit_bytes=100 * 1024 * 1024,
        ),
    )(x, W, labels2d)
    return out.reshape(T)
